# baseline (speedup 1.0000x reference)
import functools

import jax
import jax.numpy as jnp
from jax import lax
from jax.experimental import pallas as pl
from jax.experimental.pallas import tpu as pltpu
from jax.experimental.pallas import tpu_sc as plsc

CHUNK = 128


def _make_gather(n_chunks: int, vocab: int, dim: int):
    info = plsc.get_sparse_core_info()
    nc, ns = info.num_cores, info.num_subcores
    nw = nc * ns
    per_w = n_chunks // nw

    mesh = plsc.VectorSubcoreMesh(core_axis_name="c", subcore_axis_name="s")

    @functools.partial(
        pl.kernel,
        mesh=mesh,
        out_type=jax.ShapeDtypeStruct((n_chunks * CHUNK, dim), jnp.float32),
        scratch_types=[
            pltpu.VMEM((per_w * CHUNK,), jnp.int32),
            pltpu.VMEM((CHUNK, dim), jnp.float32),
            pltpu.SemaphoreType.DMA,
        ],
        compiler_params=pltpu.CompilerParams(use_tc_tiling_on_sc=False),
    )
    def gather(idx_hbm, table_hbm, out_hbm, idx_v, rows_v, sem):
        wid = lax.axis_index("s") * nc + lax.axis_index("c")
        base_chunk = wid * per_w
        pltpu.sync_copy(idx_hbm.at[pl.ds(base_chunk * CHUNK, per_w * CHUNK)], idx_v)

        def body(j, carry):
            pltpu.async_copy(
                table_hbm.at[idx_v.at[pl.ds(j * CHUNK, CHUNK)]], rows_v, sem
            ).wait()
            pltpu.sync_copy(
                rows_v, out_hbm.at[pl.ds((base_chunk + j) * CHUNK, CHUNK)]
            )
            return carry

        lax.fori_loop(0, per_w, body, 0)

    return gather


def kernel(words, table):
    b, s = words.shape
    vocab, dim = table.shape
    n = b * s
    assert n % CHUNK == 0
    n_chunks = n // CHUNK
    idx_flat = words.reshape(n).astype(jnp.int32)
    out = _make_gather(n_chunks, vocab, dim)(idx_flat, table)
    return out.reshape(b, s, dim)


# speedup vs baseline: 1.0436x; 1.0436x over previous
import functools

import jax
import jax.numpy as jnp
from jax import lax
from jax.experimental import pallas as pl
from jax.experimental.pallas import tpu as pltpu
from jax.experimental.pallas import tpu_sc as plsc

CHUNK = 128


def _make_gather(n_chunks: int, vocab: int, dim: int):
    info = plsc.get_sparse_core_info()
    nc, ns = info.num_cores, info.num_subcores
    nw = nc * ns
    per_w = n_chunks // nw

    mesh = plsc.VectorSubcoreMesh(core_axis_name="c", subcore_axis_name="s")

    nbuf = 10
    assert per_w % nbuf == 0
    n_outer = per_w // nbuf

    @functools.partial(
        pl.kernel,
        mesh=mesh,
        out_type=jax.ShapeDtypeStruct((n_chunks * CHUNK, dim), jnp.float32),
        scratch_types=[
            pltpu.VMEM((per_w * CHUNK,), jnp.int32),
            pltpu.VMEM((nbuf, CHUNK, dim), jnp.float32),
            pltpu.SemaphoreType.DMA((nbuf,)),
            pltpu.SemaphoreType.DMA((nbuf,)),
        ],
        compiler_params=pltpu.CompilerParams(use_tc_tiling_on_sc=False),
    )
    def gather(idx_hbm, table_hbm, out_hbm, idx_v, rows_v, sem_in, sem_out):
        wid = lax.axis_index("s") * nc + lax.axis_index("c")
        base_chunk = wid * per_w
        pltpu.sync_copy(idx_hbm.at[pl.ds(base_chunk * CHUNK, per_w * CHUNK)], idx_v)

        def gather_chunk(c, b):
            return pltpu.make_async_copy(
                table_hbm.at[idx_v.at[pl.ds(c * CHUNK, CHUNK)]],
                rows_v.at[b],
                sem_in.at[b],
            )

        def write_chunk(c, b):
            return pltpu.make_async_copy(
                rows_v.at[b],
                out_hbm.at[pl.ds((base_chunk + c) * CHUNK, CHUNK)],
                sem_out.at[b],
            )

        for b in range(nbuf):
            gather_chunk(b, b).start()

        def outer(g, carry):
            for b in range(nbuf):
                c = g * nbuf + b
                gather_chunk(c, b).wait()
                write_chunk(c, b).start()
                write_chunk(c, b).wait()
                gather_chunk(c + nbuf, b).start()
            return carry

        lax.fori_loop(0, n_outer - 1, outer, 0)

        for b in range(nbuf):
            c = (n_outer - 1) * nbuf + b
            gather_chunk(c, b).wait()
            write_chunk(c, b).start()
        for b in range(nbuf):
            c = (n_outer - 1) * nbuf + b
            write_chunk(c, b).wait()

    return gather


def kernel(words, table):
    b, s = words.shape
    vocab, dim = table.shape
    n = b * s
    assert n % CHUNK == 0
    n_chunks = n // CHUNK
    idx_flat = words.reshape(n).astype(jnp.int32)
    out = _make_gather(n_chunks, vocab, dim)(idx_flat, table)
    return out.reshape(b, s, dim)
